# baseline (speedup 1.0000x reference)
import functools

import jax
import jax.numpy as jnp
from jax.experimental import pallas as pl
from jax.experimental.pallas import tpu as pltpu


_SQRT_2_OVER_PI = 0.7978845608028654


def _se3d_body(x_ref, w1t_ref, w2_ref, o_ref, *, inv_n):
    xf = x_ref[0].astype(jnp.float32)
    pooled = jnp.sum(xf, axis=-1, keepdims=True) * inv_n
    h = jnp.sum(w1t_ref[...] * pooled, axis=0, keepdims=True)
    h = 0.5 * h * (1.0 + jnp.tanh(_SQRT_2_OVER_PI * (h + 0.044715 * (h * h * h))))
    g = jnp.sum(w2_ref[...] * h, axis=1, keepdims=True)
    gate = 0.5 * (1.0 + jnp.tanh(0.5 * g))
    o_ref[0] = (xf * gate).astype(jnp.bfloat16)


def kernel(x, w1, w2):
    B, C, D, H, W = x.shape
    N = D * H * W
    hidden = w1.shape[0]

    xbf = x.reshape(B, C, N).astype(jnp.bfloat16)
    w1t = jnp.transpose(w1)

    out_bf = pl.pallas_call(
        functools.partial(_se3d_body, inv_n=1.0 / N),
        out_shape=jax.ShapeDtypeStruct((B, C, N), jnp.bfloat16),
        grid=(B,),
        in_specs=[
            pl.BlockSpec((1, C, N), lambda b: (b, 0, 0)),
            pl.BlockSpec((C, hidden), lambda b: (0, 0)),
            pl.BlockSpec((C, hidden), lambda b: (0, 0)),
        ],
        out_specs=pl.BlockSpec((1, C, N), lambda b: (b, 0, 0)),
        compiler_params=pltpu.CompilerParams(
            dimension_semantics=("parallel",),
            vmem_limit_bytes=18 << 20,
        ),
    )(xbf, w1t, w2)
    return out_bf.astype(jnp.float32).reshape(B, C, D, H, W)


# speedup vs baseline: 1.0529x; 1.0529x over previous
import functools

import jax
import jax.numpy as jnp
from jax.experimental import pallas as pl
from jax.experimental.pallas import tpu as pltpu


_SQRT_2_OVER_PI = 0.7978845608028654


def _se3d_body(x_ref, w1t_ref, w2_ref, o_ref, *, inv_n):
    xf = x_ref[...].astype(jnp.float32)
    pooled = jnp.sum(xf, axis=-1, keepdims=True) * inv_n
    h = jnp.sum(w1t_ref[...][None] * pooled, axis=1, keepdims=True)
    h = 0.5 * h * (1.0 + jnp.tanh(_SQRT_2_OVER_PI * (h + 0.044715 * (h * h * h))))
    g = jnp.sum(w2_ref[...][None] * h, axis=2, keepdims=True)
    gate = 0.5 * (1.0 + jnp.tanh(0.5 * g))
    o_ref[...] = (xf * gate).astype(jnp.bfloat16)


def kernel(x, w1, w2):
    B, C, D, H, W = x.shape
    N = D * H * W
    hidden = w1.shape[0]

    xbf = x.reshape(B, C, N).astype(jnp.bfloat16)
    w1t = jnp.transpose(w1)

    out_bf = pl.pallas_call(
        functools.partial(_se3d_body, inv_n=1.0 / N),
        out_shape=jax.ShapeDtypeStruct((B, C, N), jnp.bfloat16),
        grid=(B // 2,),
        in_specs=[
            pl.BlockSpec((2, C, N), lambda b: (b, 0, 0)),
            pl.BlockSpec((C, hidden), lambda b: (0, 0)),
            pl.BlockSpec((C, hidden), lambda b: (0, 0)),
        ],
        out_specs=pl.BlockSpec((2, C, N), lambda b: (b, 0, 0)),
        compiler_params=pltpu.CompilerParams(
            dimension_semantics=("parallel",),
            vmem_limit_bytes=40 << 20,
        ),
    )(xbf, w1t, w2)
    return out_bf.astype(jnp.float32).reshape(B, C, D, H, W)


# speedup vs baseline: 1.0563x; 1.0032x over previous
import functools

import jax
import jax.numpy as jnp
from jax.experimental import pallas as pl
from jax.experimental.pallas import tpu as pltpu


_SQRT_2_OVER_PI = 0.7978845608028654


def _se3d_body(x_ref, w1t_ref, w2_ref, o_ref, *, inv_n):
    xf = x_ref[...].astype(jnp.float32)
    pooled = jnp.sum(xf, axis=-1, keepdims=True) * inv_n
    h = jnp.sum(w1t_ref[...][None] * pooled, axis=1, keepdims=True)
    h = 0.5 * h * (1.0 + jnp.tanh(_SQRT_2_OVER_PI * (h + 0.044715 * (h * h * h))))
    g = jnp.sum(w2_ref[...][None] * h, axis=2, keepdims=True)
    gate = 0.5 * (1.0 + jnp.tanh(0.5 * g))
    o_ref[...] = (xf * gate).astype(jnp.bfloat16)


def kernel(x, w1, w2):
    B, C, D, H, W = x.shape
    N = D * H * W
    hidden = w1.shape[0]

    xbf = x.reshape(B, C, N).astype(jnp.bfloat16)
    w1t = jnp.transpose(w1)

    out_bf = pl.pallas_call(
        functools.partial(_se3d_body, inv_n=1.0 / N),
        out_shape=jax.ShapeDtypeStruct((B, C, N), jnp.bfloat16),
        grid=(B // 4,),
        in_specs=[
            pl.BlockSpec((4, C, N), lambda b: (b, 0, 0)),
            pl.BlockSpec((C, hidden), lambda b: (0, 0)),
            pl.BlockSpec((C, hidden), lambda b: (0, 0)),
        ],
        out_specs=pl.BlockSpec((4, C, N), lambda b: (b, 0, 0)),
        compiler_params=pltpu.CompilerParams(
            dimension_semantics=("parallel",),
            vmem_limit_bytes=52 << 20,
        ),
    )(xbf, w1t, w2)
    return out_bf.astype(jnp.float32).reshape(B, C, D, H, W)
